# baseline (speedup 1.0000x reference)
import jax
import jax.numpy as jnp
from jax.experimental import pallas as pl
from jax.experimental.pallas import tpu as pltpu

_CH_ROWS = 256
_NBUF = 8


def _body(x_hbm, o_hbm, buf, in_sems, out_sems):
    n_rows = x_hbm.shape[0]
    n_chunks = n_rows // _CH_ROWS

    def in_copy(idx, b):
        return pltpu.make_async_copy(
            x_hbm.at[pl.ds(idx * _CH_ROWS, _CH_ROWS), :],
            buf.at[b], in_sems.at[b])

    def out_copy(idx, b):
        return pltpu.make_async_copy(
            buf.at[b],
            o_hbm.at[pl.ds(idx * _CH_ROWS, _CH_ROWS), :],
            out_sems.at[b])

    in_copy(0, 0).start()
    in_copy(1, 1).start()

    def step(idx, _):
        b = jax.lax.rem(idx, _NBUF)
        pf = jax.lax.rem(idx + 2, _NBUF)

        @pl.when(idx + 2 < n_chunks)
        def _():
            @pl.when(idx + 2 >= _NBUF)
            def _():
                out_copy(idx + 2 - _NBUF, pf).wait()
            in_copy(idx + 2, pf).start()

        in_copy(idx, b).wait()
        buf[b] = jnp.clip(buf[b], 0.0, 1.0)
        out_copy(idx, b).start()
        return ()

    jax.lax.fori_loop(0, n_chunks, step, (), unroll=False)

    for i in range(_NBUF):
        idx = n_chunks - _NBUF + i
        out_copy(idx, idx % _NBUF).wait()


def kernel(x):
    n_rows, n_cols = x.shape
    return pl.pallas_call(
        _body,
        in_specs=[pl.BlockSpec(memory_space=pl.ANY)],
        out_specs=pl.BlockSpec(memory_space=pl.ANY),
        out_shape=jax.ShapeDtypeStruct((n_rows, n_cols), x.dtype),
        scratch_shapes=[
            pltpu.VMEM((_NBUF, _CH_ROWS, n_cols), x.dtype),
            pltpu.SemaphoreType.DMA((_NBUF,)),
            pltpu.SemaphoreType.DMA((_NBUF,)),
        ],
        compiler_params=pltpu.CompilerParams(
            vmem_limit_bytes=60 * 1024 * 1024,
        ),
    )(x)


# speedup vs baseline: 1.0069x; 1.0069x over previous
import jax
import jax.numpy as jnp
from jax.experimental import pallas as pl
from jax.experimental.pallas import tpu as pltpu

_CH_ROWS = 1024
_NBUF = 4


def _body(x_hbm, o_hbm, buf, in_sems, out_sems):
    n_rows = x_hbm.shape[0]
    n_chunks = n_rows // _CH_ROWS

    def in_copy(idx, b):
        return pltpu.make_async_copy(
            x_hbm.at[pl.ds(idx * _CH_ROWS, _CH_ROWS), :],
            buf.at[b], in_sems.at[b])

    def out_copy(idx, b):
        return pltpu.make_async_copy(
            buf.at[b],
            o_hbm.at[pl.ds(idx * _CH_ROWS, _CH_ROWS), :],
            out_sems.at[b])

    in_copy(0, 0).start()
    in_copy(1, 1).start()

    def step(idx, _):
        b = jax.lax.rem(idx, _NBUF)
        pf = jax.lax.rem(idx + 2, _NBUF)

        @pl.when(idx + 2 < n_chunks)
        def _():
            @pl.when(idx + 2 >= _NBUF)
            def _():
                out_copy(idx + 2 - _NBUF, pf).wait()
            in_copy(idx + 2, pf).start()

        in_copy(idx, b).wait()
        buf[b] = jnp.clip(buf[b], 0.0, 1.0)
        out_copy(idx, b).start()
        return ()

    jax.lax.fori_loop(0, n_chunks, step, (), unroll=False)

    for i in range(_NBUF):
        idx = n_chunks - _NBUF + i
        out_copy(idx, idx % _NBUF).wait()


def kernel(x):
    n_rows, n_cols = x.shape
    return pl.pallas_call(
        _body,
        in_specs=[pl.BlockSpec(memory_space=pl.ANY)],
        out_specs=pl.BlockSpec(memory_space=pl.ANY),
        out_shape=jax.ShapeDtypeStruct((n_rows, n_cols), x.dtype),
        scratch_shapes=[
            pltpu.VMEM((_NBUF, _CH_ROWS, n_cols), x.dtype),
            pltpu.SemaphoreType.DMA((_NBUF,)),
            pltpu.SemaphoreType.DMA((_NBUF,)),
        ],
        compiler_params=pltpu.CompilerParams(
            vmem_limit_bytes=60 * 1024 * 1024,
        ),
    )(x)


# speedup vs baseline: 1.0193x; 1.0123x over previous
import jax
import jax.numpy as jnp
from jax.experimental import pallas as pl
from jax.experimental.pallas import tpu as pltpu

_CH_ROWS = 2048
_NBUF = 3


def _body(x_hbm, o_hbm, buf, in_sems, out_sems):
    n_rows = x_hbm.shape[0]
    n_chunks = n_rows // _CH_ROWS

    def in_copy(idx, b):
        return pltpu.make_async_copy(
            x_hbm.at[pl.ds(idx * _CH_ROWS, _CH_ROWS), :],
            buf.at[b], in_sems.at[b])

    def out_copy(idx, b):
        return pltpu.make_async_copy(
            buf.at[b],
            o_hbm.at[pl.ds(idx * _CH_ROWS, _CH_ROWS), :],
            out_sems.at[b])

    in_copy(0, 0).start()
    in_copy(1, 1).start()

    def step(idx, _):
        b = jax.lax.rem(idx, _NBUF)
        pf = jax.lax.rem(idx + 2, _NBUF)

        @pl.when(idx + 2 < n_chunks)
        def _():
            @pl.when(idx + 2 >= _NBUF)
            def _():
                out_copy(idx + 2 - _NBUF, pf).wait()
            in_copy(idx + 2, pf).start()

        in_copy(idx, b).wait()
        buf[b] = jnp.clip(buf[b], 0.0, 1.0)
        out_copy(idx, b).start()
        return ()

    jax.lax.fori_loop(0, n_chunks, step, (), unroll=False)

    for i in range(_NBUF):
        idx = n_chunks - _NBUF + i
        out_copy(idx, idx % _NBUF).wait()


def kernel(x):
    n_rows, n_cols = x.shape
    return pl.pallas_call(
        _body,
        in_specs=[pl.BlockSpec(memory_space=pl.ANY)],
        out_specs=pl.BlockSpec(memory_space=pl.ANY),
        out_shape=jax.ShapeDtypeStruct((n_rows, n_cols), x.dtype),
        scratch_shapes=[
            pltpu.VMEM((_NBUF, _CH_ROWS, n_cols), x.dtype),
            pltpu.SemaphoreType.DMA((_NBUF,)),
            pltpu.SemaphoreType.DMA((_NBUF,)),
        ],
        compiler_params=pltpu.CompilerParams(
            vmem_limit_bytes=60 * 1024 * 1024,
        ),
    )(x)
